# baseline (speedup 1.0000x reference)
import functools

import jax
import jax.numpy as jnp
from jax import lax
from jax.experimental import pallas as pl
from jax.experimental.pallas import tpu as pltpu
from jax.experimental.pallas import tpu_sc as plsc

N = 10000
E = 320000
D = 128
H = 8
HD = D // H
L = 2
FOUT = 240

NC = 2
NP = 2
CP = NC * NP
NS = 16
ETILE = E // NS
CH = 160
NCHUNK = ETILE // CH
NPAD = 10240

_f32 = jnp.float32


def _mlp2_body(x_ref, w1_ref, b1_ref, w2_ref, b2_ref, o_ref):
    hid = jax.nn.relu(
        jnp.dot(x_ref[...], w1_ref[...], preferred_element_type=_f32)
        + b1_ref[...])
    o_ref[...] = (
        jnp.dot(hid, w2_ref[...], preferred_element_type=_f32) + b2_ref[...])


def _mlp2(x, W1, b1, W2, b2, blk):
    R, K = x.shape
    Dh = W1.shape[1]
    Do = W2.shape[1]
    return pl.pallas_call(
        _mlp2_body,
        grid=(R // blk,),
        in_specs=[
            pl.BlockSpec((blk, K), lambda i: (i, 0)),
            pl.BlockSpec((K, Dh), lambda i: (0, 0)),
            pl.BlockSpec((1, Dh), lambda i: (0, 0)),
            pl.BlockSpec((Dh, Do), lambda i: (0, 0)),
            pl.BlockSpec((1, Do), lambda i: (0, 0)),
        ],
        out_specs=pl.BlockSpec((blk, Do), lambda i: (i, 0)),
        out_shape=jax.ShapeDtypeStruct((R, Do), _f32),
    )(x, W1, b1.reshape(1, -1), W2, b2.reshape(1, -1))


def _edge_proj_body(ea_ref, w1_ref, b1_ref, w2_ref, b2_ref, o_ref):
    hid = jax.nn.relu(
        jnp.dot(ea_ref[...], w1_ref[...], preferred_element_type=_f32)
        + b1_ref[...])
    o_ref[0] = (
        jnp.dot(hid, w2_ref[0], preferred_element_type=_f32) + b2_ref[0])


def _edge_proj(ea_pad, W1, b1, W2c, c2c, blk=2560):
    return pl.pallas_call(
        _edge_proj_body,
        grid=(CP, E // blk),
        in_specs=[
            pl.BlockSpec((blk, 8), lambda c, i: (i, 0)),
            pl.BlockSpec((8, D), lambda c, i: (0, 0)),
            pl.BlockSpec((1, D), lambda c, i: (0, 0)),
            pl.BlockSpec((1, D, 64), lambda c, i: (c, 0, 0)),
            pl.BlockSpec((1, 1, 64), lambda c, i: (c, 0, 0)),
        ],
        out_specs=pl.BlockSpec((1, blk, 64), lambda c, i: (c, i, 0)),
        out_shape=jax.ShapeDtypeStruct((CP, E, 64), _f32),
    )(ea_pad, W1, b1.reshape(1, -1), W2c, c2c).reshape(CP * E, 64)


def _node_proj_body(h_ref, wq_ref, wkv_ref, q_ref, kv_ref):
    hh = h_ref[...]
    q_ref[0] = jnp.dot(hh, wq_ref[0], preferred_element_type=_f32)
    kv_ref[0] = jnp.dot(hh, wkv_ref[0], preferred_element_type=_f32)


def _node_proj(h, Wq_arr, Wkv_arr, blk=2560):
    q, kv = pl.pallas_call(
        _node_proj_body,
        grid=(CP, NPAD // blk),
        in_specs=[
            pl.BlockSpec((blk, D), lambda c, i: (i, 0)),
            pl.BlockSpec((1, D, 32), lambda c, i: (c, 0, 0)),
            pl.BlockSpec((1, D, 64), lambda c, i: (c, 0, 0)),
        ],
        out_specs=[
            pl.BlockSpec((1, blk, 32), lambda c, i: (c, i, 0)),
            pl.BlockSpec((1, blk, 64), lambda c, i: (c, i, 0)),
        ],
        out_shape=[
            jax.ShapeDtypeStruct((CP, NPAD, 32), _f32),
            jax.ShapeDtypeStruct((CP, NPAD, 64), _f32),
        ],
    )(h, Wq_arr, Wkv_arr)
    return q.reshape(CP * NPAD, 32), kv.reshape(CP * NPAD, 64)


def _update_body(t_ref, h_ref, wo_ref, w1_ref, b1_ref, w2_ref, b2_ref,
                 o_ref):
    pieces = []
    for hh in range(H):
        cp = hh // 2
        hl = hh % 2
        rec = 1.0 / (t_ref[cp, :, 32 + hl:33 + hl] + 1e-9)
        pieces.append(t_ref[cp, :, hl * HD:(hl + 1) * HD] * rec)
    aggn = jnp.concatenate(pieces, axis=1)
    h1 = h_ref[...] + jnp.dot(aggn, wo_ref[...], preferred_element_type=_f32)
    hid = jax.nn.relu(
        jnp.dot(h1, w1_ref[...], preferred_element_type=_f32) + b1_ref[...])
    o_ref[...] = h1 + (
        jnp.dot(hid, w2_ref[...], preferred_element_type=_f32) + b2_ref[...])


def _node_update(ttab, h, Wo_l, Wf1, bf1, Wf2, bf2, blk=2560):
    return pl.pallas_call(
        _update_body,
        grid=(NPAD // blk,),
        in_specs=[
            pl.BlockSpec((CP, blk, 64), lambda i: (0, i, 0)),
            pl.BlockSpec((blk, D), lambda i: (i, 0)),
            pl.BlockSpec((D, D), lambda i: (0, 0)),
            pl.BlockSpec((D, D), lambda i: (0, 0)),
            pl.BlockSpec((1, D), lambda i: (0, 0)),
            pl.BlockSpec((D, D), lambda i: (0, 0)),
            pl.BlockSpec((1, D), lambda i: (0, 0)),
        ],
        out_specs=pl.BlockSpec((blk, D), lambda i: (i, 0)),
        out_shape=jax.ShapeDtypeStruct((NPAD, D), _f32),
    )(ttab, h, Wo_l, Wf1, bf1.reshape(1, -1), Wf2, bf2.reshape(1, -1))


NV = NPAD // 2
NTA = NV // NS


def _sc_attention(ekv_cat, src, dst, qh_cat, kvh_cat):
    mesh = plsc.VectorSubcoreMesh(core_axis_name="c", subcore_axis_name="s")

    @functools.partial(
        pl.kernel,
        mesh=mesh,
        compiler_params=pltpu.CompilerParams(use_tc_tiling_on_sc=False),
        out_type=jax.ShapeDtypeStruct((CP * NV, D), _f32),
        scratch_types=[
            pltpu.VMEM((CH,), jnp.int32),
            pltpu.VMEM((CH,), jnp.int32),
            pltpu.VMEM((CH,), jnp.int32),
            pltpu.VMEM((CH,), jnp.int32),
            pltpu.VMEM((CH * 16,), jnp.int32),
            pltpu.VMEM((CH, 32), _f32),
            pltpu.VMEM((CH, 64), _f32),
            pltpu.VMEM((CH, 64), _f32),
            pltpu.VMEM((CH, D), _f32),
            pltpu.VMEM_SHARED((NV, D), _f32),
            pltpu.SemaphoreType.DMA,
            pltpu.SemaphoreType.DMA,
        ],
    )
    def k(ekv_hbm, src_hbm, dst_hbm, q_hbm, kv_hbm, out_hbm,
          sbuf, dbuf, abuf, r2buf, parb, qbuf, kvbuf, ebuf,
          cbv, acc_s, sem1, sem2):
        cid = lax.axis_index("c")
        sid = lax.axis_index("s")
        lane = lax.iota(jnp.int32, 16)
        zero16 = jnp.zeros((16,), _f32)
        gdn = lax.GatherDimensionNumbers(
            offset_dims=(), collapsed_slice_dims=(0,), start_index_map=(0,))
        perms = [jnp.bitwise_xor(lane, kk)[:, None] for kk in (8, 4, 2, 1)]
        lconst = [jnp.full((16, 1), t, jnp.int32) for t in range(16)]

        def bcast_sum(v):
            for p in perms:
                v = v + lax.gather(v, p, gdn, slice_sizes=(1,),
                                   mode=lax.GatherScatterMode.PROMISE_IN_BOUNDS)
            return v

        def bcast_lane(v, t):
            return lax.gather(v, lconst[t], gdn, slice_sizes=(1,),
                              mode=lax.GatherScatterMode.PROMISE_IN_BOUNDS)

        def zrow(j, _):
            for c8 in range(D // 16):
                cbv[j, pl.ds(c8 * 16, 16)] = zero16
            return 0

        for p in range(NP):
            cp0 = jnp.broadcast_to(cid * NP + p, (16,)).astype(jnp.int32)
            tab0 = cp0 * NPAD
            lax.fori_loop(0, CH, zrow, 0)

            def zslab(i, _):
                pltpu.sync_copy(cbv, acc_s.at[pl.ds(sid * NTA + i * CH, CH)])
                return 0
            lax.fori_loop(0, NTA // CH, zslab, 0)
            plsc.subcore_barrier()

            def chunk_body(i, _):
                base = sid * ETILE + i * CH
                pltpu.sync_copy(src_hbm.at[pl.ds(base, CH)], sbuf)
                pltpu.sync_copy(dst_hbm.at[pl.ds(base, CH)], dbuf)

                def adj(j16, _):
                    sl = pl.ds(j16 * 16, 16)
                    dvec = dbuf[sl]
                    sbuf[sl] = sbuf[sl] + tab0
                    abuf[sl] = dvec + tab0
                    r2buf[sl] = lax.shift_right_logical(dvec, 1)
                    dpar = jnp.bitwise_and(dvec, 1)
                    for t in range(16):
                        parb[pl.ds((j16 * 16 + t) * 16, 16)] = (
                            bcast_lane(dpar, t))
                    return 0
                lax.fori_loop(0, CH // 16, adj, 0)

                cp_q = pltpu.async_copy(q_hbm.at[abuf], qbuf, sem1)
                cp_kv = pltpu.async_copy(kv_hbm.at[sbuf], kvbuf, sem2)
                ebase = (cid * NP + p) * E + base
                pltpu.sync_copy(ekv_hbm.at[pl.ds(ebase, CH)], ebuf)
                cp_q.wait()
                cp_kv.wait()

                def edge_body(j2, _):
                    for u in range(2):
                        j = j2 * 2 + u
                        pv = parb[pl.ds(j * 16, 16)]
                        meven = pv == 0
                        ex_acc = zero16
                        for hl in range(2):
                            sk = pl.ds(hl * HD, 16)
                            sv = pl.ds(32 + hl * HD, 16)
                            qv = qbuf[j, sk]
                            kvec = kvbuf[j, sk] + ebuf[j, sk]
                            exb = jnp.exp(bcast_sum(qv * kvec))
                            vvec = (kvbuf[j, sv] + ebuf[j, sv]) * exb
                            cbv[j, sk] = jnp.where(meven, vvec, 0.0)
                            cbv[j, pl.ds(64 + hl * HD, 16)] = (
                                jnp.where(meven, 0.0, vvec))
                            ex_acc = jnp.where(lane == hl, exb, ex_acc)
                        exm = jnp.where(lane < 2, ex_acc, 0.0)
                        cbv[j, pl.ds(32, 16)] = jnp.where(meven, exm, 0.0)
                        cbv[j, pl.ds(96, 16)] = jnp.where(meven, 0.0, exm)
                    return 0
                lax.fori_loop(0, CH // 2, edge_body, 0)
                pltpu.sync_copy(cbv, acc_s.at[r2buf], add=True)
                return 0
            lax.fori_loop(0, NCHUNK, chunk_body, 0)
            plsc.subcore_barrier()
            pltpu.sync_copy(
                acc_s.at[pl.ds(sid * NTA, NTA)],
                out_hbm.at[pl.ds((cid * NP + p) * NV + sid * NTA, NTA)])

    return k(ekv_cat, src, dst, qh_cat, kvh_cat)


def kernel(x, edge_index, edge_attr, W_in1, b_in1, W_in2, b_in2,
           W_e1, b_e1, W_e2, b_e2, Wq, Wk, Wv, Wo,
           W_ff1, b_ff1, W_ff2, b_ff2, W_out1, b_out1, W_out2, b_out2):
    src = edge_index[0]
    dst = edge_index[1]

    x_pad = jnp.pad(x, ((0, NPAD - N), (0, 4)))
    W_in1p = jnp.pad(W_in1, ((0, 4), (0, 0)))
    ea_pad = jnp.pad(edge_attr, ((0, 0), (0, 5)))
    W_e1p = jnp.pad(W_e1, ((0, 5), (0, 0)))

    h = _mlp2(x_pad, W_in1p, b_in1, W_in2, b_in2, blk=2560)

    for l in range(L):
        ek_w = W_e2 @ Wk[l]
        ev_w = W_e2 @ Wv[l]
        ek_b = b_e2 @ Wk[l]
        ev_b = b_e2 @ Wv[l]
        hp = 2 * HD
        W2c = jnp.stack([
            jnp.concatenate([ek_w[:, c * hp:(c + 1) * hp],
                             ev_w[:, c * hp:(c + 1) * hp]], axis=1)
            for c in range(CP)])
        c2c = jnp.stack([
            jnp.concatenate([ek_b[c * hp:(c + 1) * hp],
                             ev_b[c * hp:(c + 1) * hp]]).reshape(1, -1)
            for c in range(CP)])
        ekv_cat = _edge_proj(ea_pad, W_e1p, b_e1, W2c, c2c)

        Wq_s = Wq[l] * 0.25
        Wq_arr = jnp.stack([Wq_s[:, c * hp:(c + 1) * hp] for c in range(CP)])
        Wkv_arr = jnp.stack([
            jnp.concatenate([Wk[l][:, c * hp:(c + 1) * hp],
                             Wv[l][:, c * hp:(c + 1) * hp]], axis=1)
            for c in range(CP)])
        qh_cat, kvh_cat = _node_proj(h, Wq_arr, Wkv_arr)

        out_a = _sc_attention(ekv_cat, src, dst, qh_cat, kvh_cat)
        ttab = out_a.reshape(CP, NV, 2, 64).reshape(CP, NPAD, 64)
        h = _node_update(ttab, h, Wo[l], W_ff1[l],
                         b_ff1[l], W_ff2[l], b_ff2[l])

    return _mlp2(h, W_out1, b_out1, W_out2, b_out2, blk=2560)[:N]


# speedup vs baseline: 1.6621x; 1.6621x over previous
import functools

import jax
import jax.numpy as jnp
from jax import lax
from jax.experimental import pallas as pl
from jax.experimental.pallas import tpu as pltpu
from jax.experimental.pallas import tpu_sc as plsc

N = 10000
E = 320000
D = 128
H = 8
HD = D // H
L = 2
FOUT = 240

NC = 2
NP = 2
CP = NC * NP
NS = 16
ETILE = E // NS
CH = 160
NCHUNK = ETILE // CH
NPAD = 10240

_f32 = jnp.float32


def _mlp2_body(x_ref, w1_ref, b1_ref, w2_ref, b2_ref, o_ref):
    hid = jax.nn.relu(
        jnp.dot(x_ref[...], w1_ref[...], preferred_element_type=_f32)
        + b1_ref[...])
    o_ref[...] = (
        jnp.dot(hid, w2_ref[...], preferred_element_type=_f32) + b2_ref[...])


def _mlp2(x, W1, b1, W2, b2, blk):
    R, K = x.shape
    Dh = W1.shape[1]
    Do = W2.shape[1]
    return pl.pallas_call(
        _mlp2_body,
        grid=(R // blk,),
        in_specs=[
            pl.BlockSpec((blk, K), lambda i: (i, 0)),
            pl.BlockSpec((K, Dh), lambda i: (0, 0)),
            pl.BlockSpec((1, Dh), lambda i: (0, 0)),
            pl.BlockSpec((Dh, Do), lambda i: (0, 0)),
            pl.BlockSpec((1, Do), lambda i: (0, 0)),
        ],
        out_specs=pl.BlockSpec((blk, Do), lambda i: (i, 0)),
        out_shape=jax.ShapeDtypeStruct((R, Do), _f32),
    )(x, W1, b1.reshape(1, -1), W2, b2.reshape(1, -1))


def _edge_proj_body(ea_ref, w1_ref, b1_ref, w2_ref, b2_ref, o_ref):
    hid = jax.nn.relu(
        jnp.dot(ea_ref[...], w1_ref[...], preferred_element_type=_f32)
        + b1_ref[...])
    o_ref[0] = (
        jnp.dot(hid, w2_ref[0], preferred_element_type=_f32) + b2_ref[0])


def _edge_proj(ea_pad, W1, b1, W2c, c2c, blk=2560):
    return pl.pallas_call(
        _edge_proj_body,
        grid=(CP, E // blk),
        in_specs=[
            pl.BlockSpec((blk, 8), lambda c, i: (i, 0)),
            pl.BlockSpec((8, D), lambda c, i: (0, 0)),
            pl.BlockSpec((1, D), lambda c, i: (0, 0)),
            pl.BlockSpec((1, D, 64), lambda c, i: (c, 0, 0)),
            pl.BlockSpec((1, 1, 64), lambda c, i: (c, 0, 0)),
        ],
        out_specs=pl.BlockSpec((1, blk, 64), lambda c, i: (c, i, 0)),
        out_shape=jax.ShapeDtypeStruct((CP, E, 64), _f32),
    )(ea_pad, W1, b1.reshape(1, -1), W2c, c2c).reshape(CP * E, 64)


def _node_proj_body(h_ref, wq_ref, wkv_ref, q_ref, kv_ref):
    hh = h_ref[...]
    q_ref[0] = jnp.dot(hh, wq_ref[0], preferred_element_type=_f32)
    kv_ref[0] = jnp.dot(hh, wkv_ref[0], preferred_element_type=_f32)


def _node_proj(h, Wq_arr, Wkv_arr, blk=2560):
    q, kv = pl.pallas_call(
        _node_proj_body,
        grid=(CP, NPAD // blk),
        in_specs=[
            pl.BlockSpec((blk, D), lambda c, i: (i, 0)),
            pl.BlockSpec((1, D, D), lambda c, i: (c, 0, 0)),
            pl.BlockSpec((1, D, D), lambda c, i: (c, 0, 0)),
        ],
        out_specs=[
            pl.BlockSpec((1, blk, D), lambda c, i: (c, i, 0)),
            pl.BlockSpec((1, blk, D), lambda c, i: (c, i, 0)),
        ],
        out_shape=[
            jax.ShapeDtypeStruct((CP, NPAD, D), _f32),
            jax.ShapeDtypeStruct((CP, NPAD, D), _f32),
        ],
    )(h, Wq_arr, Wkv_arr)
    return q.reshape(CP * NPAD, D), kv.reshape(CP * NPAD, D)


def _update_body(t_ref, h_ref, wo_ref, w1_ref, b1_ref, w2_ref, b2_ref,
                 o_ref):
    pieces = []
    for hh in range(H):
        cp = hh // 2
        hl = hh % 2
        rec = 1.0 / (t_ref[cp, :, 32 + hl:33 + hl] + 1e-9)
        pieces.append(t_ref[cp, :, hl * HD:(hl + 1) * HD] * rec)
    aggn = jnp.concatenate(pieces, axis=1)
    h1 = h_ref[...] + jnp.dot(aggn, wo_ref[...], preferred_element_type=_f32)
    hid = jax.nn.relu(
        jnp.dot(h1, w1_ref[...], preferred_element_type=_f32) + b1_ref[...])
    o_ref[...] = h1 + (
        jnp.dot(hid, w2_ref[...], preferred_element_type=_f32) + b2_ref[...])


def _node_update(ttab, h, Wo_l, Wf1, bf1, Wf2, bf2, blk=2560):
    return pl.pallas_call(
        _update_body,
        grid=(NPAD // blk,),
        in_specs=[
            pl.BlockSpec((CP, blk, 64), lambda i: (0, i, 0)),
            pl.BlockSpec((blk, D), lambda i: (i, 0)),
            pl.BlockSpec((D, D), lambda i: (0, 0)),
            pl.BlockSpec((D, D), lambda i: (0, 0)),
            pl.BlockSpec((1, D), lambda i: (0, 0)),
            pl.BlockSpec((D, D), lambda i: (0, 0)),
            pl.BlockSpec((1, D), lambda i: (0, 0)),
        ],
        out_specs=pl.BlockSpec((blk, D), lambda i: (i, 0)),
        out_shape=jax.ShapeDtypeStruct((NPAD, D), _f32),
    )(ttab, h, Wo_l, Wf1, bf1.reshape(1, -1), Wf2, bf2.reshape(1, -1))


NV = NPAD // 2
NTA = NV // NS


def _sc_attention(ekv_cat, src, dst, qh_cat, kvh_cat):
    mesh = plsc.VectorSubcoreMesh(core_axis_name="c", subcore_axis_name="s")

    @functools.partial(
        pl.kernel,
        mesh=mesh,
        out_type=jax.ShapeDtypeStruct((CP * NV, D), _f32),
        scratch_types=[
            pltpu.VMEM((CH,), jnp.int32),
            pltpu.VMEM((CH,), jnp.int32),
            pltpu.VMEM((CH,), jnp.int32),
            pltpu.VMEM((CH,), jnp.int32),
            pltpu.VMEM((CH * 16,), jnp.int32),
            pltpu.VMEM((CH, D), _f32),
            pltpu.VMEM((CH, D), _f32),
            pltpu.VMEM((CH, 64), _f32),
            pltpu.VMEM((CH, D), _f32),
            pltpu.VMEM_SHARED((NV, D), _f32),
            pltpu.SemaphoreType.DMA,
            pltpu.SemaphoreType.DMA,
        ],
    )
    def k(ekv_hbm, src_hbm, dst_hbm, q_hbm, kv_hbm, out_hbm,
          sbuf, dbuf, abuf, r2buf, parb, qbuf, kvbuf, ebuf,
          cbv, acc_s, sem1, sem2):
        cid = lax.axis_index("c")
        sid = lax.axis_index("s")
        lane = lax.iota(jnp.int32, 16)
        zero16 = jnp.zeros((16,), _f32)
        gdn = lax.GatherDimensionNumbers(
            offset_dims=(), collapsed_slice_dims=(0,), start_index_map=(0,))
        perms = [jnp.bitwise_xor(lane, kk)[:, None] for kk in (8, 4, 2, 1)]
        lconst = [jnp.full((16, 1), t, jnp.int32) for t in range(16)]

        def bcast_sum(v):
            for p in perms:
                v = v + lax.gather(v, p, gdn, slice_sizes=(1,),
                                   mode=lax.GatherScatterMode.PROMISE_IN_BOUNDS)
            return v

        def bcast_lane(v, t):
            return lax.gather(v, lconst[t], gdn, slice_sizes=(1,),
                              mode=lax.GatherScatterMode.PROMISE_IN_BOUNDS)

        def zrow(j, _):
            for c8 in range(D // 16):
                cbv[j, pl.ds(c8 * 16, 16)] = zero16
            return 0

        for p in range(NP):
            cp0 = jnp.broadcast_to(cid * NP + p, (16,)).astype(jnp.int32)
            tab0 = cp0 * NPAD
            lax.fori_loop(0, CH, zrow, 0)

            def zslab(i, _):
                pltpu.sync_copy(cbv, acc_s.at[pl.ds(sid * NTA + i * CH, CH)])
                return 0
            lax.fori_loop(0, NTA // CH, zslab, 0)
            plsc.subcore_barrier()

            def chunk_body(i, _):
                base = sid * ETILE + i * CH
                pltpu.sync_copy(src_hbm.at[pl.ds(base, CH)], sbuf)
                pltpu.sync_copy(dst_hbm.at[pl.ds(base, CH)], dbuf)

                def adj(j16, _):
                    sl = pl.ds(j16 * 16, 16)
                    dvec = dbuf[sl]
                    sbuf[sl] = sbuf[sl] + tab0
                    abuf[sl] = dvec + tab0
                    r2buf[sl] = lax.shift_right_logical(dvec, 1)
                    dpar = jnp.bitwise_and(dvec, 1)
                    for t in range(16):
                        parb[pl.ds((j16 * 16 + t) * 16, 16)] = (
                            bcast_lane(dpar, t))
                    return 0
                lax.fori_loop(0, CH // 16, adj, 0)

                cp_q = pltpu.async_copy(q_hbm.at[abuf], qbuf, sem1)
                cp_kv = pltpu.async_copy(kv_hbm.at[sbuf], kvbuf, sem2)
                ebase = (cid * NP + p) * E + base
                pltpu.sync_copy(ekv_hbm.at[pl.ds(ebase, CH)], ebuf)
                cp_q.wait()
                cp_kv.wait()

                def edge_body(j2, _):
                    for u in range(2):
                        j = j2 * 2 + u
                        pv = parb[pl.ds(j * 16, 16)]
                        meven = pv == 0
                        ex_acc = zero16
                        for hl in range(2):
                            sk = pl.ds(hl * HD, 16)
                            sv = pl.ds(32 + hl * HD, 16)
                            qv = qbuf[j, sk]
                            kvec = kvbuf[j, sk] + ebuf[j, sk]
                            exb = jnp.exp(bcast_sum(qv * kvec))
                            vvec = (kvbuf[j, sv] + ebuf[j, sv]) * exb
                            cbv[j, sk] = jnp.where(meven, vvec, 0.0)
                            cbv[j, pl.ds(64 + hl * HD, 16)] = (
                                jnp.where(meven, 0.0, vvec))
                            ex_acc = jnp.where(lane == hl, exb, ex_acc)
                        exm = jnp.where(lane < 2, ex_acc, 0.0)
                        cbv[j, pl.ds(32, 16)] = jnp.where(meven, exm, 0.0)
                        cbv[j, pl.ds(96, 16)] = jnp.where(meven, 0.0, exm)
                    return 0
                lax.fori_loop(0, CH // 2, edge_body, 0)
                pltpu.sync_copy(cbv, acc_s.at[r2buf], add=True)
                return 0
            lax.fori_loop(0, NCHUNK, chunk_body, 0)
            plsc.subcore_barrier()
            pltpu.sync_copy(
                acc_s.at[pl.ds(sid * NTA, NTA)],
                out_hbm.at[pl.ds((cid * NP + p) * NV + sid * NTA, NTA)])

    return k(ekv_cat, src, dst, qh_cat, kvh_cat)


def kernel(x, edge_index, edge_attr, W_in1, b_in1, W_in2, b_in2,
           W_e1, b_e1, W_e2, b_e2, Wq, Wk, Wv, Wo,
           W_ff1, b_ff1, W_ff2, b_ff2, W_out1, b_out1, W_out2, b_out2):
    src = edge_index[0]
    dst = edge_index[1]

    x_pad = jnp.pad(x, ((0, NPAD - N), (0, 4)))
    W_in1p = jnp.pad(W_in1, ((0, 4), (0, 0)))
    ea_pad = jnp.pad(edge_attr, ((0, 0), (0, 5)))
    W_e1p = jnp.pad(W_e1, ((0, 5), (0, 0)))

    h = _mlp2(x_pad, W_in1p, b_in1, W_in2, b_in2, blk=2560)

    for l in range(L):
        ek_w = W_e2 @ Wk[l]
        ev_w = W_e2 @ Wv[l]
        ek_b = b_e2 @ Wk[l]
        ev_b = b_e2 @ Wv[l]
        hp = 2 * HD
        W2c = jnp.stack([
            jnp.concatenate([ek_w[:, c * hp:(c + 1) * hp],
                             ev_w[:, c * hp:(c + 1) * hp]], axis=1)
            for c in range(CP)])
        c2c = jnp.stack([
            jnp.concatenate([ek_b[c * hp:(c + 1) * hp],
                             ev_b[c * hp:(c + 1) * hp]]).reshape(1, -1)
            for c in range(CP)])
        ekv_cat = _edge_proj(ea_pad, W_e1p, b_e1, W2c, c2c)

        Wq_s = Wq[l] * 0.25
        Wq_arr = jnp.stack([
            jnp.concatenate([Wq_s[:, c * hp:(c + 1) * hp]] * 4, axis=1)
            for c in range(CP)])
        Wkv_arr = jnp.stack([
            jnp.concatenate([Wk[l][:, c * hp:(c + 1) * hp],
                             Wv[l][:, c * hp:(c + 1) * hp]] * 2, axis=1)
            for c in range(CP)])
        qh_cat, kvh_cat = _node_proj(h, Wq_arr, Wkv_arr)

        out_a = _sc_attention(ekv_cat, src, dst, qh_cat, kvh_cat)
        ttab = out_a.reshape(CP, NV, 2, 64).reshape(CP, NPAD, 64)
        h = _node_update(ttab, h, Wo[l], W_ff1[l],
                         b_ff1[l], W_ff2[l], b_ff2[l])

    return _mlp2(h, W_out1, b_out1, W_out2, b_out2, blk=2560)[:N]


# speedup vs baseline: 1.7214x; 1.0357x over previous
import functools

import jax
import jax.numpy as jnp
from jax import lax
from jax.experimental import pallas as pl
from jax.experimental.pallas import tpu as pltpu
from jax.experimental.pallas import tpu_sc as plsc

N = 10000
E = 320000
D = 128
H = 8
HD = D // H
L = 2
FOUT = 240

NC = 2
NP = 2
CP = NC * NP
NS = 16
ETILE = E // NS
CH = 80
NCHUNK = ETILE // CH
NPAD = 10240

_f32 = jnp.float32


def _mlp2_body(x_ref, w1_ref, b1_ref, w2_ref, b2_ref, o_ref):
    hid = jax.nn.relu(
        jnp.dot(x_ref[...], w1_ref[...], preferred_element_type=_f32)
        + b1_ref[...])
    o_ref[...] = (
        jnp.dot(hid, w2_ref[...], preferred_element_type=_f32) + b2_ref[...])


def _mlp2(x, W1, b1, W2, b2, blk):
    R, K = x.shape
    Dh = W1.shape[1]
    Do = W2.shape[1]
    return pl.pallas_call(
        _mlp2_body,
        grid=(R // blk,),
        in_specs=[
            pl.BlockSpec((blk, K), lambda i: (i, 0)),
            pl.BlockSpec((K, Dh), lambda i: (0, 0)),
            pl.BlockSpec((1, Dh), lambda i: (0, 0)),
            pl.BlockSpec((Dh, Do), lambda i: (0, 0)),
            pl.BlockSpec((1, Do), lambda i: (0, 0)),
        ],
        out_specs=pl.BlockSpec((blk, Do), lambda i: (i, 0)),
        out_shape=jax.ShapeDtypeStruct((R, Do), _f32),
    )(x, W1, b1.reshape(1, -1), W2, b2.reshape(1, -1))


def _edge_proj_body(ea_ref, w1_ref, b1_ref, w2_ref, b2_ref, o_ref):
    hid = jax.nn.relu(
        jnp.dot(ea_ref[...], w1_ref[...], preferred_element_type=_f32)
        + b1_ref[...])
    o_ref[0] = (
        jnp.dot(hid, w2_ref[0], preferred_element_type=_f32) + b2_ref[0])


def _edge_proj(ea_pad, W1, b1, W2c, c2c, blk=2560):
    return pl.pallas_call(
        _edge_proj_body,
        grid=(CP, E // blk),
        in_specs=[
            pl.BlockSpec((blk, 8), lambda c, i: (i, 0)),
            pl.BlockSpec((8, D), lambda c, i: (0, 0)),
            pl.BlockSpec((1, D), lambda c, i: (0, 0)),
            pl.BlockSpec((1, D, 64), lambda c, i: (c, 0, 0)),
            pl.BlockSpec((1, 1, 64), lambda c, i: (c, 0, 0)),
        ],
        out_specs=pl.BlockSpec((1, blk, 64), lambda c, i: (c, i, 0)),
        out_shape=jax.ShapeDtypeStruct((CP, E, 64), _f32),
    )(ea_pad, W1, b1.reshape(1, -1), W2c, c2c).reshape(CP * E, 64)


def _node_proj_body(h_ref, wq_ref, wkv_ref, q_ref, kv_ref):
    hh = h_ref[...]
    q_ref[0] = jnp.dot(hh, wq_ref[0], preferred_element_type=_f32)
    kv_ref[0] = jnp.dot(hh, wkv_ref[0], preferred_element_type=_f32)


def _node_proj(h, Wq_arr, Wkv_arr, blk=2560):
    q, kv = pl.pallas_call(
        _node_proj_body,
        grid=(CP, NPAD // blk),
        in_specs=[
            pl.BlockSpec((blk, D), lambda c, i: (i, 0)),
            pl.BlockSpec((1, D, D), lambda c, i: (c, 0, 0)),
            pl.BlockSpec((1, D, D), lambda c, i: (c, 0, 0)),
        ],
        out_specs=[
            pl.BlockSpec((1, blk, D), lambda c, i: (c, i, 0)),
            pl.BlockSpec((1, blk, D), lambda c, i: (c, i, 0)),
        ],
        out_shape=[
            jax.ShapeDtypeStruct((CP, NPAD, D), _f32),
            jax.ShapeDtypeStruct((CP, NPAD, D), _f32),
        ],
    )(h, Wq_arr, Wkv_arr)
    return q.reshape(CP * NPAD, D), kv.reshape(CP * NPAD, D)


def _update_body(t_ref, h_ref, wo_ref, w1_ref, b1_ref, w2_ref, b2_ref,
                 o_ref):
    pieces = []
    for hh in range(H):
        cp = hh // 2
        hl = hh % 2
        rec = 1.0 / (t_ref[cp, :, 32 + hl:33 + hl] + 1e-9)
        pieces.append(t_ref[cp, :, hl * HD:(hl + 1) * HD] * rec)
    aggn = jnp.concatenate(pieces, axis=1)
    h1 = h_ref[...] + jnp.dot(aggn, wo_ref[...], preferred_element_type=_f32)
    hid = jax.nn.relu(
        jnp.dot(h1, w1_ref[...], preferred_element_type=_f32) + b1_ref[...])
    o_ref[...] = h1 + (
        jnp.dot(hid, w2_ref[...], preferred_element_type=_f32) + b2_ref[...])


def _node_update(ttab, h, Wo_l, Wf1, bf1, Wf2, bf2, blk=2560):
    return pl.pallas_call(
        _update_body,
        grid=(NPAD // blk,),
        in_specs=[
            pl.BlockSpec((CP, blk, 64), lambda i: (0, i, 0)),
            pl.BlockSpec((blk, D), lambda i: (i, 0)),
            pl.BlockSpec((D, D), lambda i: (0, 0)),
            pl.BlockSpec((D, D), lambda i: (0, 0)),
            pl.BlockSpec((1, D), lambda i: (0, 0)),
            pl.BlockSpec((D, D), lambda i: (0, 0)),
            pl.BlockSpec((1, D), lambda i: (0, 0)),
        ],
        out_specs=pl.BlockSpec((blk, D), lambda i: (i, 0)),
        out_shape=jax.ShapeDtypeStruct((NPAD, D), _f32),
    )(ttab, h, Wo_l, Wf1, bf1.reshape(1, -1), Wf2, bf2.reshape(1, -1))


NV = NPAD // 2
NTA = NV // NS


def _sc_attention(ekv_cat, src, dst, qh_cat, kvh_cat):
    mesh = plsc.VectorSubcoreMesh(core_axis_name="c", subcore_axis_name="s")

    @functools.partial(
        pl.kernel,
        mesh=mesh,
        out_type=jax.ShapeDtypeStruct((CP * NV, D), _f32),
        scratch_types=[
            [pltpu.VMEM((CH,), jnp.int32)] * 2,
            [pltpu.VMEM((CH,), jnp.int32)] * 2,
            [pltpu.VMEM((CH,), jnp.int32)] * 2,
            [pltpu.VMEM((CH,), jnp.int32)] * 2,
            [pltpu.VMEM((CH * 16,), jnp.int32)] * 2,
            [pltpu.VMEM((CH, D), _f32)] * 2,
            [pltpu.VMEM((CH, D), _f32)] * 2,
            [pltpu.VMEM((CH, 64), _f32)] * 2,
            pltpu.VMEM((CH, D), _f32),
            pltpu.VMEM_SHARED((NV, D), _f32),
            [pltpu.SemaphoreType.DMA] * 2,
            [pltpu.SemaphoreType.DMA] * 2,
        ],
    )
    def k(ekv_hbm, src_hbm, dst_hbm, q_hbm, kv_hbm, out_hbm,
          sbufs, dbufs, abufs, r2bufs, parbs, qbufs, kvbufs, ebufs,
          cbv, acc_s, sem_i, sem_g):
        cid = lax.axis_index("c")
        sid = lax.axis_index("s")
        lane = lax.iota(jnp.int32, 16)
        zero16 = jnp.zeros((16,), _f32)
        gdn = lax.GatherDimensionNumbers(
            offset_dims=(), collapsed_slice_dims=(0,), start_index_map=(0,))
        perms = [jnp.bitwise_xor(lane, kk)[:, None] for kk in (8, 4, 2, 1)]
        lconst = [jnp.full((16, 1), t, jnp.int32) for t in range(16)]

        def bcast_sum(v):
            for p in perms:
                v = v + lax.gather(v, p, gdn, slice_sizes=(1,),
                                   mode=lax.GatherScatterMode.PROMISE_IN_BOUNDS)
            return v

        def bcast_lane(v, t):
            return lax.gather(v, lconst[t], gdn, slice_sizes=(1,),
                              mode=lax.GatherScatterMode.PROMISE_IN_BOUNDS)

        def zrow(j, _):
            for c8 in range(D // 16):
                cbv[j, pl.ds(c8 * 16, 16)] = zero16
            return 0

        for p in range(NP):
            cp0 = jnp.broadcast_to(cid * NP + p, (16,)).astype(jnp.int32)
            tab0 = cp0 * NPAD
            lax.fori_loop(0, CH, zrow, 0)

            def zslab(i, _):
                pltpu.sync_copy(cbv, acc_s.at[pl.ds(sid * NTA + i * CH, CH)])
                return 0
            lax.fori_loop(0, NTA // CH, zslab, 0)
            plsc.subcore_barrier()

            def stage_a(i, b):
                base = sid * ETILE + i * CH
                ebase = (cid * NP + p) * E + base
                pltpu.async_copy(src_hbm.at[pl.ds(base, CH)], sbufs[b],
                                 sem_i[b])
                pltpu.async_copy(dst_hbm.at[pl.ds(base, CH)], dbufs[b],
                                 sem_i[b])
                pltpu.async_copy(ekv_hbm.at[pl.ds(ebase, CH)], ebufs[b],
                                 sem_i[b])

            def stage_b(i, b):
                base = sid * ETILE + i * CH
                ebase = (cid * NP + p) * E + base
                pltpu.make_async_copy(src_hbm.at[pl.ds(base, CH)], sbufs[b],
                                      sem_i[b]).wait()
                pltpu.make_async_copy(dst_hbm.at[pl.ds(base, CH)], dbufs[b],
                                      sem_i[b]).wait()
                pltpu.make_async_copy(ekv_hbm.at[pl.ds(ebase, CH)], ebufs[b],
                                      sem_i[b]).wait()

                def adj(j16, _):
                    sl = pl.ds(j16 * 16, 16)
                    dvec = dbufs[b][sl]
                    sbufs[b][sl] = sbufs[b][sl] + tab0
                    abufs[b][sl] = dvec + tab0
                    r2bufs[b][sl] = lax.shift_right_logical(dvec, 1)
                    dpar = jnp.bitwise_and(dvec, 1)
                    for t in range(16):
                        parbs[b][pl.ds((j16 * 16 + t) * 16, 16)] = (
                            bcast_lane(dpar, t))
                    return 0
                lax.fori_loop(0, CH // 16, adj, 0)
                pltpu.async_copy(q_hbm.at[abufs[b]], qbufs[b], sem_g[b])
                pltpu.async_copy(kv_hbm.at[sbufs[b]], kvbufs[b], sem_g[b])

            def stage_c(b):
                pltpu.make_async_copy(q_hbm.at[abufs[b]], qbufs[b],
                                      sem_g[b]).wait()
                pltpu.make_async_copy(kv_hbm.at[sbufs[b]], kvbufs[b],
                                      sem_g[b]).wait()

                def edge_body(j2, _):
                    for u in range(2):
                        j = j2 * 2 + u
                        pv = parbs[b][pl.ds(j * 16, 16)]
                        meven = pv == 0
                        ex_acc = zero16
                        for hl in range(2):
                            sk = pl.ds(hl * HD, 16)
                            sv = pl.ds(32 + hl * HD, 16)
                            qv = qbufs[b][j, sk]
                            kvec = kvbufs[b][j, sk] + ebufs[b][j, sk]
                            exb = jnp.exp(bcast_sum(qv * kvec))
                            vvec = (kvbufs[b][j, sv] + ebufs[b][j, sv]) * exb
                            cbv[j, sk] = jnp.where(meven, vvec, 0.0)
                            cbv[j, pl.ds(64 + hl * HD, 16)] = (
                                jnp.where(meven, 0.0, vvec))
                            ex_acc = jnp.where(lane == hl, exb, ex_acc)
                        exm = jnp.where(lane < 2, ex_acc, 0.0)
                        cbv[j, pl.ds(32, 16)] = jnp.where(meven, exm, 0.0)
                        cbv[j, pl.ds(96, 16)] = jnp.where(meven, 0.0, exm)
                    return 0
                lax.fori_loop(0, CH // 2, edge_body, 0)
                pltpu.sync_copy(cbv, acc_s.at[r2bufs[b]], add=True)

            stage_a(0, 0)

            def outer(g, _):
                i0 = 2 * g
                stage_b(i0, 0)
                stage_a(i0 + 1, 1)
                stage_c(0)
                stage_b(i0 + 1, 1)
                stage_a(i0 + 2, 0)
                stage_c(1)
                return 0
            lax.fori_loop(0, NCHUNK // 2 - 1, outer, 0)
            stage_b(NCHUNK - 2, 0)
            stage_a(NCHUNK - 1, 1)
            stage_c(0)
            stage_b(NCHUNK - 1, 1)
            stage_c(1)
            plsc.subcore_barrier()
            pltpu.sync_copy(
                acc_s.at[pl.ds(sid * NTA, NTA)],
                out_hbm.at[pl.ds((cid * NP + p) * NV + sid * NTA, NTA)])

    return k(ekv_cat, src, dst, qh_cat, kvh_cat)


def kernel(x, edge_index, edge_attr, W_in1, b_in1, W_in2, b_in2,
           W_e1, b_e1, W_e2, b_e2, Wq, Wk, Wv, Wo,
           W_ff1, b_ff1, W_ff2, b_ff2, W_out1, b_out1, W_out2, b_out2):
    src = edge_index[0]
    dst = edge_index[1]

    x_pad = jnp.pad(x, ((0, NPAD - N), (0, 4)))
    W_in1p = jnp.pad(W_in1, ((0, 4), (0, 0)))
    ea_pad = jnp.pad(edge_attr, ((0, 0), (0, 5)))
    W_e1p = jnp.pad(W_e1, ((0, 5), (0, 0)))

    h = _mlp2(x_pad, W_in1p, b_in1, W_in2, b_in2, blk=2560)

    for l in range(L):
        ek_w = W_e2 @ Wk[l]
        ev_w = W_e2 @ Wv[l]
        ek_b = b_e2 @ Wk[l]
        ev_b = b_e2 @ Wv[l]
        hp = 2 * HD
        W2c = jnp.stack([
            jnp.concatenate([ek_w[:, c * hp:(c + 1) * hp],
                             ev_w[:, c * hp:(c + 1) * hp]], axis=1)
            for c in range(CP)])
        c2c = jnp.stack([
            jnp.concatenate([ek_b[c * hp:(c + 1) * hp],
                             ev_b[c * hp:(c + 1) * hp]]).reshape(1, -1)
            for c in range(CP)])
        ekv_cat = _edge_proj(ea_pad, W_e1p, b_e1, W2c, c2c)

        Wq_s = Wq[l] * 0.25
        Wq_arr = jnp.stack([
            jnp.concatenate([Wq_s[:, c * hp:(c + 1) * hp]] * 4, axis=1)
            for c in range(CP)])
        Wkv_arr = jnp.stack([
            jnp.concatenate([Wk[l][:, c * hp:(c + 1) * hp],
                             Wv[l][:, c * hp:(c + 1) * hp]] * 2, axis=1)
            for c in range(CP)])
        qh_cat, kvh_cat = _node_proj(h, Wq_arr, Wkv_arr)

        out_a = _sc_attention(ekv_cat, src, dst, qh_cat, kvh_cat)
        ttab = out_a.reshape(CP, NV, 2, 64).reshape(CP, NPAD, 64)
        h = _node_update(ttab, h, Wo[l], W_ff1[l],
                         b_ff1[l], W_ff2[l], b_ff2[l])

    return _mlp2(h, W_out1, b_out1, W_out2, b_out2, blk=2560)[:N]


# speedup vs baseline: 1.8900x; 1.0980x over previous
import functools

import jax
import jax.numpy as jnp
from jax import lax
from jax.experimental import pallas as pl
from jax.experimental.pallas import tpu as pltpu
from jax.experimental.pallas import tpu_sc as plsc

N = 10000
E = 320000
D = 128
H = 8
HD = D // H
L = 2
FOUT = 240

NC = 2
NP = 2
CP = NC * NP
NS = 16
ETILE = E // NS
CH = 80
NCHUNK = ETILE // CH
NPAD = 10240

_f32 = jnp.float32


def _mlp2_body(x_ref, w1_ref, b1_ref, w2_ref, b2_ref, o_ref):
    hid = jax.nn.relu(
        jnp.dot(x_ref[...], w1_ref[...], preferred_element_type=_f32)
        + b1_ref[...])
    o_ref[...] = (
        jnp.dot(hid, w2_ref[...], preferred_element_type=_f32) + b2_ref[...])


def _mlp2(x, W1, b1, W2, b2, blk):
    R, K = x.shape
    Dh = W1.shape[1]
    Do = W2.shape[1]
    return pl.pallas_call(
        _mlp2_body,
        grid=(R // blk,),
        in_specs=[
            pl.BlockSpec((blk, K), lambda i: (i, 0)),
            pl.BlockSpec((K, Dh), lambda i: (0, 0)),
            pl.BlockSpec((1, Dh), lambda i: (0, 0)),
            pl.BlockSpec((Dh, Do), lambda i: (0, 0)),
            pl.BlockSpec((1, Do), lambda i: (0, 0)),
        ],
        out_specs=pl.BlockSpec((blk, Do), lambda i: (i, 0)),
        out_shape=jax.ShapeDtypeStruct((R, Do), _f32),
    )(x, W1, b1.reshape(1, -1), W2, b2.reshape(1, -1))


def _edge_proj_body(ea_ref, w1_ref, b1_ref, w2_ref, b2_ref, o_ref):
    hid = jax.nn.relu(
        jnp.dot(ea_ref[...], w1_ref[...], preferred_element_type=_f32)
        + b1_ref[...])
    o_ref[0] = (
        jnp.dot(hid, w2_ref[0], preferred_element_type=_f32) + b2_ref[0])


def _edge_proj(ea_pad, W1, b1, W2c, c2c, blk=2560):
    return pl.pallas_call(
        _edge_proj_body,
        grid=(CP, E // blk),
        in_specs=[
            pl.BlockSpec((blk, 8), lambda c, i: (i, 0)),
            pl.BlockSpec((8, D), lambda c, i: (0, 0)),
            pl.BlockSpec((1, D), lambda c, i: (0, 0)),
            pl.BlockSpec((1, D, 64), lambda c, i: (c, 0, 0)),
            pl.BlockSpec((1, 1, 64), lambda c, i: (c, 0, 0)),
        ],
        out_specs=pl.BlockSpec((1, blk, 64), lambda c, i: (c, i, 0)),
        out_shape=jax.ShapeDtypeStruct((CP, E, 64), _f32),
    )(ea_pad, W1, b1.reshape(1, -1), W2c, c2c).reshape(CP * E, 64)


def _node_proj_body(h_ref, wq_ref, wkv_ref, q_ref, kv_ref):
    hh = h_ref[...]
    q_ref[0] = jnp.dot(hh, wq_ref[0], preferred_element_type=_f32)
    kv_ref[0] = jnp.dot(hh, wkv_ref[0], preferred_element_type=_f32)


def _node_proj(h, Wq_arr, Wkv_arr, blk=2560):
    q, kv = pl.pallas_call(
        _node_proj_body,
        grid=(CP, NPAD // blk),
        in_specs=[
            pl.BlockSpec((blk, D), lambda c, i: (i, 0)),
            pl.BlockSpec((1, D, D), lambda c, i: (c, 0, 0)),
            pl.BlockSpec((1, D, D), lambda c, i: (c, 0, 0)),
        ],
        out_specs=[
            pl.BlockSpec((1, blk, D), lambda c, i: (c, i, 0)),
            pl.BlockSpec((1, blk, D), lambda c, i: (c, i, 0)),
        ],
        out_shape=[
            jax.ShapeDtypeStruct((CP, NPAD, D), _f32),
            jax.ShapeDtypeStruct((CP, NPAD, D), _f32),
        ],
    )(h, Wq_arr, Wkv_arr)
    return q.reshape(CP * NPAD, D), kv.reshape(CP * NPAD, D)


def _update_body(t_ref, h_ref, wo_ref, w1_ref, b1_ref, w2_ref, b2_ref,
                 o_ref):
    pieces = []
    for hh in range(H):
        cp = hh // 2
        hl = hh % 2
        rec = 1.0 / (t_ref[cp, :, 32 + hl:33 + hl] + 1e-9)
        pieces.append(t_ref[cp, :, hl * HD:(hl + 1) * HD] * rec)
    aggn = jnp.concatenate(pieces, axis=1)
    h1 = h_ref[...] + jnp.dot(aggn, wo_ref[...], preferred_element_type=_f32)
    hid = jax.nn.relu(
        jnp.dot(h1, w1_ref[...], preferred_element_type=_f32) + b1_ref[...])
    o_ref[...] = h1 + (
        jnp.dot(hid, w2_ref[...], preferred_element_type=_f32) + b2_ref[...])


def _node_update(ttab, h, Wo_l, Wf1, bf1, Wf2, bf2, blk=2560):
    return pl.pallas_call(
        _update_body,
        grid=(NPAD // blk,),
        in_specs=[
            pl.BlockSpec((CP, blk, 64), lambda i: (0, i, 0)),
            pl.BlockSpec((blk, D), lambda i: (i, 0)),
            pl.BlockSpec((D, D), lambda i: (0, 0)),
            pl.BlockSpec((D, D), lambda i: (0, 0)),
            pl.BlockSpec((1, D), lambda i: (0, 0)),
            pl.BlockSpec((D, D), lambda i: (0, 0)),
            pl.BlockSpec((1, D), lambda i: (0, 0)),
        ],
        out_specs=pl.BlockSpec((blk, D), lambda i: (i, 0)),
        out_shape=jax.ShapeDtypeStruct((NPAD, D), _f32),
    )(ttab, h, Wo_l, Wf1, bf1.reshape(1, -1), Wf2, bf2.reshape(1, -1))


NV = NPAD // 2
NTA = NV // NS


def _sc_attention(ekv_cat, src, dst, qh_cat, kvh_cat):
    mesh = plsc.VectorSubcoreMesh(core_axis_name="c", subcore_axis_name="s")

    @functools.partial(
        pl.kernel,
        mesh=mesh,
        compiler_params=pltpu.CompilerParams(needs_layout_passes=False),
        out_type=jax.ShapeDtypeStruct((CP * NV, D), _f32),
        scratch_types=[
            [pltpu.VMEM((CH,), jnp.int32)] * 2,
            [pltpu.VMEM((CH,), jnp.int32)] * 2,
            [pltpu.VMEM((CH,), jnp.int32)] * 2,
            [pltpu.VMEM((CH,), jnp.int32)] * 2,
            [pltpu.VMEM((CH * 16,), jnp.int32)] * 2,
            [pltpu.VMEM((CH, D), _f32)] * 2,
            [pltpu.VMEM((CH, D), _f32)] * 2,
            [pltpu.VMEM((CH, 64), _f32)] * 2,
            [pltpu.VMEM((CH, D), _f32)] * 2,
            pltpu.VMEM_SHARED((NV, D), _f32),
            [pltpu.SemaphoreType.DMA] * 2,
            [pltpu.SemaphoreType.DMA] * 2,
            [pltpu.SemaphoreType.DMA] * 2,
        ],
    )
    def k(ekv_hbm, src_hbm, dst_hbm, q_hbm, kv_hbm, out_hbm,
          sbufs, dbufs, abufs, r2bufs, parbs, qbufs, kvbufs, ebufs,
          cbvs, acc_s, sem_i, sem_g, sem_s):
        cid = lax.axis_index("c")
        sid = lax.axis_index("s")
        lane = lax.iota(jnp.int32, 16)
        zero16 = jnp.zeros((16,), _f32)
        gdn = lax.GatherDimensionNumbers(
            offset_dims=(), collapsed_slice_dims=(0,), start_index_map=(0,))
        perms = [jnp.bitwise_xor(lane, kk)[:, None] for kk in (8, 4, 2, 1)]
        lconst = [jnp.full((16, 1), t, jnp.int32) for t in range(16)]

        def bcast_sum(v):
            for p in perms:
                v = v + lax.gather(v, p, gdn, slice_sizes=(1,),
                                   mode=lax.GatherScatterMode.PROMISE_IN_BOUNDS)
            return v

        def bcast_lane(v, t):
            return lax.gather(v, lconst[t], gdn, slice_sizes=(1,),
                              mode=lax.GatherScatterMode.PROMISE_IN_BOUNDS)

        def zrow(j, _):
            for c8 in range(D // 16):
                cbvs[0][j, pl.ds(c8 * 16, 16)] = zero16
                cbvs[1][j, pl.ds(c8 * 16, 16)] = zero16
            return 0

        zi16 = jnp.zeros((16,), jnp.int32)

        def zidx(i, _):
            r2bufs[0][pl.ds(i * 16, 16)] = zi16
            r2bufs[1][pl.ds(i * 16, 16)] = zi16
            return 0
        lax.fori_loop(0, CH // 16, zidx, 0)

        for p in range(NP):
            cp0 = jnp.broadcast_to(cid * NP + p, (16,)).astype(jnp.int32)
            tab0 = cp0 * NPAD
            lax.fori_loop(0, CH, zrow, 0)

            def zslab(i, _):
                pltpu.sync_copy(cbvs[0],
                                acc_s.at[pl.ds(sid * NTA + i * CH, CH)])
                return 0
            lax.fori_loop(0, NTA // CH, zslab, 0)
            plsc.subcore_barrier()
            for b0 in range(2):
                pltpu.async_copy(cbvs[b0], acc_s.at[r2bufs[b0]], sem_s[b0],
                                 add=True)

            def stage_a(i, b):
                base = sid * ETILE + i * CH
                ebase = (cid * NP + p) * E + base
                pltpu.async_copy(src_hbm.at[pl.ds(base, CH)], sbufs[b],
                                 sem_i[b])
                pltpu.async_copy(dst_hbm.at[pl.ds(base, CH)], dbufs[b],
                                 sem_i[b])
                pltpu.async_copy(ekv_hbm.at[pl.ds(ebase, CH)], ebufs[b],
                                 sem_i[b])

            def stage_b(i, b):
                pltpu.make_async_copy(cbvs[b], acc_s.at[r2bufs[b]],
                                      sem_s[b]).wait()
                base = sid * ETILE + i * CH
                ebase = (cid * NP + p) * E + base
                pltpu.make_async_copy(src_hbm.at[pl.ds(base, CH)], sbufs[b],
                                      sem_i[b]).wait()
                pltpu.make_async_copy(dst_hbm.at[pl.ds(base, CH)], dbufs[b],
                                      sem_i[b]).wait()
                pltpu.make_async_copy(ekv_hbm.at[pl.ds(ebase, CH)], ebufs[b],
                                      sem_i[b]).wait()

                def adj(j16, _):
                    sl = pl.ds(j16 * 16, 16)
                    dvec = dbufs[b][sl]
                    sbufs[b][sl] = sbufs[b][sl] + tab0
                    abufs[b][sl] = dvec + tab0
                    r2bufs[b][sl] = lax.shift_right_logical(dvec, 1)
                    dpar = jnp.bitwise_and(dvec, 1)
                    for t in range(16):
                        parbs[b][pl.ds((j16 * 16 + t) * 16, 16)] = (
                            bcast_lane(dpar, t))
                    return 0
                lax.fori_loop(0, CH // 16, adj, 0)
                pltpu.async_copy(q_hbm.at[abufs[b]], qbufs[b], sem_g[b])
                pltpu.async_copy(kv_hbm.at[sbufs[b]], kvbufs[b], sem_g[b])

            def stage_c(cb):
                pltpu.make_async_copy(q_hbm.at[abufs[cb]], qbufs[cb],
                                      sem_g[cb]).wait()
                pltpu.make_async_copy(kv_hbm.at[sbufs[cb]], kvbufs[cb],
                                      sem_g[cb]).wait()

                def edge_body(j2, _):
                    for u in range(2):
                        j = j2 * 2 + u
                        pv = parbs[cb][pl.ds(j * 16, 16)]
                        meven = pv == 0
                        ex_acc = zero16
                        for hl in range(2):
                            sk = pl.ds(hl * HD, 16)
                            sv = pl.ds(32 + hl * HD, 16)
                            qv = qbufs[cb][j, sk]
                            kvec = kvbufs[cb][j, sk] + ebufs[cb][j, sk]
                            exb = jnp.exp(bcast_sum(qv * kvec))
                            vvec = (kvbufs[cb][j, sv] + ebufs[cb][j, sv]) * exb
                            cbvs[cb][j, sk] = jnp.where(meven, vvec, 0.0)
                            cbvs[cb][j, pl.ds(64 + hl * HD, 16)] = (
                                jnp.where(meven, 0.0, vvec))
                            ex_acc = jnp.where(lane == hl, exb, ex_acc)
                        exm = jnp.where(lane < 2, ex_acc, 0.0)
                        cbvs[cb][j, pl.ds(32, 16)] = jnp.where(meven, exm, 0.0)
                        cbvs[cb][j, pl.ds(96, 16)] = jnp.where(meven, 0.0, exm)
                    return 0
                lax.fori_loop(0, CH // 2, edge_body, 0)
                pltpu.async_copy(cbvs[cb], acc_s.at[r2bufs[cb]], sem_s[cb],
                                 add=True)

            stage_a(0, 0)

            def outer(g, _):
                i0 = 2 * g
                stage_b(i0, 0)
                stage_a(i0 + 1, 1)
                stage_c(0)
                stage_b(i0 + 1, 1)
                stage_a(i0 + 2, 0)
                stage_c(1)
                return 0
            lax.fori_loop(0, NCHUNK // 2 - 1, outer, 0)
            stage_b(NCHUNK - 2, 0)
            stage_a(NCHUNK - 1, 1)
            stage_c(0)
            stage_b(NCHUNK - 1, 1)
            stage_c(1)
            for b0 in range(2):
                pltpu.make_async_copy(cbvs[b0], acc_s.at[r2bufs[b0]],
                                      sem_s[b0]).wait()
            plsc.subcore_barrier()
            pltpu.sync_copy(
                acc_s.at[pl.ds(sid * NTA, NTA)],
                out_hbm.at[pl.ds((cid * NP + p) * NV + sid * NTA, NTA)])

    return k(ekv_cat, src, dst, qh_cat, kvh_cat)


def kernel(x, edge_index, edge_attr, W_in1, b_in1, W_in2, b_in2,
           W_e1, b_e1, W_e2, b_e2, Wq, Wk, Wv, Wo,
           W_ff1, b_ff1, W_ff2, b_ff2, W_out1, b_out1, W_out2, b_out2):
    src = edge_index[0]
    dst = edge_index[1]

    x_pad = jnp.pad(x, ((0, NPAD - N), (0, 4)))
    W_in1p = jnp.pad(W_in1, ((0, 4), (0, 0)))
    ea_pad = jnp.pad(edge_attr, ((0, 0), (0, 5)))
    W_e1p = jnp.pad(W_e1, ((0, 5), (0, 0)))

    h = _mlp2(x_pad, W_in1p, b_in1, W_in2, b_in2, blk=2560)

    for l in range(L):
        ek_w = W_e2 @ Wk[l]
        ev_w = W_e2 @ Wv[l]
        ek_b = b_e2 @ Wk[l]
        ev_b = b_e2 @ Wv[l]
        hp = 2 * HD
        W2c = jnp.stack([
            jnp.concatenate([ek_w[:, c * hp:(c + 1) * hp],
                             ev_w[:, c * hp:(c + 1) * hp]], axis=1)
            for c in range(CP)])
        c2c = jnp.stack([
            jnp.concatenate([ek_b[c * hp:(c + 1) * hp],
                             ev_b[c * hp:(c + 1) * hp]]).reshape(1, -1)
            for c in range(CP)])
        ekv_cat = _edge_proj(ea_pad, W_e1p, b_e1, W2c, c2c)

        Wq_s = Wq[l] * 0.25
        Wq_arr = jnp.stack([
            jnp.concatenate([Wq_s[:, c * hp:(c + 1) * hp]] * 4, axis=1)
            for c in range(CP)])
        Wkv_arr = jnp.stack([
            jnp.concatenate([Wk[l][:, c * hp:(c + 1) * hp],
                             Wv[l][:, c * hp:(c + 1) * hp]] * 2, axis=1)
            for c in range(CP)])
        qh_cat, kvh_cat = _node_proj(h, Wq_arr, Wkv_arr)

        out_a = _sc_attention(ekv_cat, src, dst, qh_cat, kvh_cat)
        ttab = out_a.reshape(CP, NV, 2, 64).reshape(CP, NPAD, 64)
        h = _node_update(ttab, h, Wo[l], W_ff1[l],
                         b_ff1[l], W_ff2[l], b_ff2[l])

    return _mlp2(h, W_out1, b_out1, W_out2, b_out2, blk=2560)[:N]
